# baseline (speedup 1.0000x reference)
import functools
import jax
import jax.numpy as jnp
from jax import lax
from jax.experimental import pallas as pl
from jax.experimental.pallas import tpu as pltpu
from jax.experimental.pallas import tpu_sc as plsc

POOL = 7
NPIX = POOL * POOL
PIXPAD = 56
CH = 256
OUTROW = NPIX * CH
NBOX = 1000
NBOXPAD = 1024
BOX_PER_W = 32
BOXREC = 8
CLS = 81
CLSPAD = 96
CLSHALF = 4
IDXBUF = 4 * PIXPAD + 8
A0 = (224.0 / 512.0) ** 2 / 2.0
A1 = (224.0 / 512.0) ** 2 * 2.0
A2 = (224.0 / 512.0) ** 2 * 8.0
B3, B4, B5, B6 = 0, 4096, 5120, 5376


def _roialign_call(table, boxesf, clsf):
    mesh = plsc.VectorSubcoreMesh(core_axis_name="c", subcore_axis_name="s")

    @functools.partial(
        pl.kernel,
        out_type=jax.ShapeDtypeStruct((NBOX, NPIX, CH), jnp.float32),
        mesh=mesh,
        scratch_types=[
            pltpu.VMEM((BOX_PER_W * BOXREC,), jnp.float32),
            pltpu.VMEM((CLSHALF * CLSPAD,), jnp.float32),
            pltpu.VMEM((IDXBUF,), jnp.int32),
            pltpu.VMEM((IDXBUF,), jnp.int32),
            pltpu.VMEM((4, PIXPAD, CH), jnp.float32),
            pltpu.VMEM((4, PIXPAD, CH), jnp.float32),
            pltpu.VMEM((NPIX, CH), jnp.float32),
            pltpu.VMEM((256,), jnp.float32),
            pltpu.VMEM((256,), jnp.float32),
            pltpu.SemaphoreType.DMA,
            pltpu.SemaphoreType.DMA,
            pltpu.SemaphoreType.DMA,
        ],
    )
    def sc_kernel(table_h, boxes_h, cls_h, out_h,
                  boxv, clsv, idx0, idx1, gv0, gv1, ov, wb0, wb1,
                  gsem0, gsem1, osem):
        bufs = ((idx0, gv0, gsem0, wb0), (idx1, gv1, gsem1, wb1))
        wid = lax.axis_index("s") * 2 + lax.axis_index("c")
        base_box = wid * BOX_PER_W
        nb = jnp.minimum(BOX_PER_W, NBOX - base_box)

        pltpu.sync_copy(
            boxes_h.at[pl.ds(base_box * BOXREC, BOX_PER_W * BOXREC)], boxv)
        pltpu.sync_copy(
            cls_h.at[pl.ds(base_box * CLSPAD, CLSHALF * CLSPAD)], clsv)

        lanes = lax.iota(jnp.int32, 16)
        lane0 = lanes == 0
        neg_inf = jnp.full((16,), -jnp.inf, jnp.float32)
        perms = [lanes ^ k for k in (1, 2, 4, 8)]

        def prep_box(i, b):
            idxv = bufs[b][0]
            wbuf = bufs[b][3]
            bv = boxv[pl.ds(i * BOXREC, 16)]
            y1 = bv[0]
            x1 = bv[1]
            y2 = bv[2]
            x2 = bv[3]
            dy = y2 - y1
            dx = x2 - x1
            area = dy * dx
            l3 = area < A0
            l4 = area < A1
            l5 = area < A2
            hm1f = jnp.where(l3, 63.0, jnp.where(l4, 31.0,
                             jnp.where(l5, 15.0, 7.0))).astype(jnp.float32)
            hm1i = jnp.where(l3, 63, jnp.where(l4, 31, jnp.where(l5, 15, 7)))
            wdim = jnp.where(l3, 64, jnp.where(l4, 32, jnp.where(l5, 16, 8)))
            base = jnp.where(l3, B3, jnp.where(l4, B4, jnp.where(l5, B5, B6)))

            cb = (i & (CLSHALF - 1)) * CLSPAD
            c0 = clsv[pl.ds(cb, 16)]
            m = jnp.where(lane0, neg_inf, c0)
            for cchunk in range(1, 6):
                m = jnp.maximum(m, clsv[pl.ds(cb + 16 * cchunk, 16)])
            for perm in perms:
                m = jnp.maximum(m, jnp.take(m, perm))
            valid = (m[0] > c0[0]).astype(jnp.float32)

            wvecs = []
            for c in (3, 0, 1, 2):
                p = lanes + 16 * c
                pi = (p * 9363) >> 16
                pj = p - pi * POOL
                ly = pi.astype(jnp.float32) * jnp.float32(1.0 / 6.0)
                lx = pj.astype(jnp.float32) * jnp.float32(1.0 / 6.0)
                ys = (y1 + dy * ly) * hm1f
                xs = (x1 + dx * lx) * hm1f
                y0f = ys.astype(jnp.int32)
                x0f = xs.astype(jnp.int32)
                wy = ys - y0f.astype(jnp.float32)
                wx = xs - x0f.astype(jnp.float32)
                y0 = jnp.clip(y0f, 0, hm1i)
                x0 = jnp.clip(x0f, 0, hm1i)
                y1i = jnp.clip(y0f + 1, 0, hm1i)
                x1i = jnp.clip(x0f + 1, 0, hm1i)
                rbase = base + y0 * wdim
                rbase1 = base + y1i * wdim
                corners = (rbase + x0, rbase + x1i, rbase1 + x0, rbase1 + x1i)
                for k in range(4):
                    idxv[pl.ds(PIXPAD * k + 16 * c, 16)] = corners[k]
                wyv = wy * valid
                w11 = wyv * wx
                w10 = wyv - w11
                wxv = wx * valid
                w01 = wxv - w11
                w00 = valid - wyv - w01
                wbuf[pl.ds(64 * c, 16)] = w00
                wbuf[pl.ds(64 * c + 16, 16)] = w01
                wbuf[pl.ds(64 * c + 32, 16)] = w10
                wbuf[pl.ds(64 * c + 48, 16)] = w11

        def fire_gathers(b):
            idxv, gvb, gsem, _ = bufs[b]
            for k in range(4):
                pltpu.async_copy(
                    table_h.at[idxv.at[pl.ds(PIXPAD * k, PIXPAD)]],
                    gvb.at[k], gsem)

        def wait_gathers(b):
            idxv, gvb, gsem, _ = bufs[b]
            for k in range(4):
                pltpu.make_async_copy(
                    table_h.at[idxv.at[pl.ds(PIXPAD * k, PIXPAD)]],
                    gvb.at[k], gsem).wait()

        def blend(b):
            gvb = bufs[b][1]
            wbuf = bufs[b][3]
            for c in range(4):
                w00v = wbuf[pl.ds(64 * c, 16)]
                w01v = wbuf[pl.ds(64 * c + 16, 16)]
                w10v = wbuf[pl.ds(64 * c + 32, 16)]
                w11v = wbuf[pl.ds(64 * c + 48, 16)]
                for l in range(16):
                    p = 16 * c + l
                    if p >= NPIX:
                        break
                    w00 = w00v[l]
                    w01 = w01v[l]
                    w10 = w10v[l]
                    w11 = w11v[l]

                    @plsc.parallel_loop(0, CH, 16, unroll=4)
                    def _chunk(s0, p=p, w00=w00, w01=w01, w10=w10, w11=w11):
                        s = pl.ds(s0, 16)
                        ov[p, s] = (
                            gvb[0, p, s] * w00 + gvb[1, p, s] * w01
                            + gvb[2, p, s] * w10 + gvb[3, p, s] * w11)

        def fire_out(i):
            pltpu.async_copy(ov, out_h.at[base_box + i], osem)

        def wait_out(i):
            pltpu.make_async_copy(ov, out_h.at[base_box + i], osem).wait()

        prep_box(0, 0)
        fire_gathers(0)

        def pair_body(j, carry):
            i0 = 2 * j
            i1 = i0 + 1
            prep_box(i1, 1)
            fire_gathers(1)
            wait_gathers(0)

            @pl.when(j >= 1)
            def _():
                wait_out(i0 - 1)
            blend(0)
            fire_out(i0)

            @pl.when(jnp.logical_and((i0 + 2) & (CLSHALF - 1) == 0,
                                     i0 + 2 < nb))
            def _():
                pltpu.sync_copy(
                    cls_h.at[pl.ds((base_box + i0 + 2) * CLSPAD,
                                   CLSHALF * CLSPAD)], clsv)

            @pl.when(i0 + 2 < nb)
            def _():
                prep_box(i0 + 2, 0)
                fire_gathers(0)
            wait_gathers(1)
            wait_out(i0)
            blend(1)
            fire_out(i1)
            return carry

        lax.fori_loop(0, nb // 2, pair_body, 0)
        wait_out(nb - 1)

    return sc_kernel(table, boxesf, clsf)


def kernel(P3, P4, P5, P6, boxes, nms_classification):
    table = jnp.concatenate(
        [P3.reshape(4096, CH), P4.reshape(1024, CH),
         P5.reshape(256, CH), P6.reshape(64, CH)], axis=0)
    boxesf = jnp.pad(boxes.reshape(NBOX, 4),
                     ((0, NBOXPAD - NBOX), (0, BOXREC - 4))).reshape(
                         NBOXPAD * BOXREC)
    cls = nms_classification.reshape(NBOX, CLS)
    clsf = jnp.pad(cls, ((0, NBOXPAD - NBOX), (0, CLSPAD - CLS)),
                   constant_values=-jnp.inf).reshape(NBOXPAD * CLSPAD)
    out = _roialign_call(table, boxesf, clsf)
    return out.reshape(1, NBOX, POOL, POOL, CH)


# speedup vs baseline: 1.0497x; 1.0497x over previous
import functools
import jax
import jax.numpy as jnp
from jax import lax
from jax.experimental import pallas as pl
from jax.experimental.pallas import tpu as pltpu
from jax.experimental.pallas import tpu_sc as plsc

POOL = 7
NPIX = POOL * POOL
PIXPAD = 56
CH = 256
OUTROW = NPIX * CH
NBOX = 1000
NBOXPAD = 1024
BOX_PER_W = 32
BOXREC = 8
CLS = 81
CLSPAD = 96
CLSHALF = 4
IDXBUF = 4 * PIXPAD + 8
A0 = (224.0 / 512.0) ** 2 / 2.0
A1 = (224.0 / 512.0) ** 2 * 2.0
A2 = (224.0 / 512.0) ** 2 * 8.0
B3, B4, B5, B6 = 0, 4096, 5120, 5376


def _roialign_call(table, boxesf, clsf):
    mesh = plsc.VectorSubcoreMesh(core_axis_name="c", subcore_axis_name="s")

    @functools.partial(
        pl.kernel,
        out_type=jax.ShapeDtypeStruct((NBOX, NPIX, CH), jnp.float32),
        mesh=mesh,
        scratch_types=[
            pltpu.VMEM((BOX_PER_W * BOXREC,), jnp.float32),
            pltpu.VMEM((CLSHALF * CLSPAD,), jnp.float32),
            pltpu.VMEM((IDXBUF,), jnp.int32),
            pltpu.VMEM((IDXBUF,), jnp.int32),
            pltpu.VMEM((4 * PIXPAD, CH), jnp.float32),
            pltpu.VMEM((4 * PIXPAD, CH), jnp.float32),
            pltpu.VMEM((NPIX, CH), jnp.float32),
            pltpu.VMEM((256,), jnp.float32),
            pltpu.VMEM((256,), jnp.float32),
            pltpu.SemaphoreType.DMA,
            pltpu.SemaphoreType.DMA,
            pltpu.SemaphoreType.DMA,
        ],
    )
    def sc_kernel(table_h, boxes_h, cls_h, out_h,
                  boxv, clsv, idx0, idx1, gv0, gv1, ov, wb0, wb1,
                  gsem0, gsem1, osem):
        bufs = ((idx0, gv0, gsem0, wb0), (idx1, gv1, gsem1, wb1))
        wid = lax.axis_index("s") * 2 + lax.axis_index("c")
        base_box = wid * BOX_PER_W
        nb = jnp.minimum(BOX_PER_W, NBOX - base_box)

        pltpu.sync_copy(
            boxes_h.at[pl.ds(base_box * BOXREC, BOX_PER_W * BOXREC)], boxv)
        pltpu.sync_copy(
            cls_h.at[pl.ds(base_box * CLSPAD, CLSHALF * CLSPAD)], clsv)

        lanes = lax.iota(jnp.int32, 16)
        lane0 = lanes == 0
        neg_inf = jnp.full((16,), -jnp.inf, jnp.float32)
        perms = [lanes ^ k for k in (1, 2, 4, 8)]

        def prep_box(i, b):
            idxv = bufs[b][0]
            wbuf = bufs[b][3]
            bv = boxv[pl.ds(i * BOXREC, 16)]
            y1 = bv[0]
            x1 = bv[1]
            y2 = bv[2]
            x2 = bv[3]
            dy = y2 - y1
            dx = x2 - x1
            area = dy * dx
            l3 = area < A0
            l4 = area < A1
            l5 = area < A2
            hm1f = jnp.where(l3, 63.0, jnp.where(l4, 31.0,
                             jnp.where(l5, 15.0, 7.0))).astype(jnp.float32)
            hm1i = jnp.where(l3, 63, jnp.where(l4, 31, jnp.where(l5, 15, 7)))
            wdim = jnp.where(l3, 64, jnp.where(l4, 32, jnp.where(l5, 16, 8)))
            base = jnp.where(l3, B3, jnp.where(l4, B4, jnp.where(l5, B5, B6)))

            cb = (i & (CLSHALF - 1)) * CLSPAD
            c0 = clsv[pl.ds(cb, 16)]
            m = jnp.where(lane0, neg_inf, c0)
            for cchunk in range(1, 6):
                m = jnp.maximum(m, clsv[pl.ds(cb + 16 * cchunk, 16)])
            for perm in perms:
                m = jnp.maximum(m, jnp.take(m, perm))
            valid = (m[0] > c0[0]).astype(jnp.float32)

            wvecs = []
            for c in (3, 0, 1, 2):
                p = lanes + 16 * c
                pi = (p * 9363) >> 16
                pj = p - pi * POOL
                ly = pi.astype(jnp.float32) * jnp.float32(1.0 / 6.0)
                lx = pj.astype(jnp.float32) * jnp.float32(1.0 / 6.0)
                ys = (y1 + dy * ly) * hm1f
                xs = (x1 + dx * lx) * hm1f
                y0f = ys.astype(jnp.int32)
                x0f = xs.astype(jnp.int32)
                wy = ys - y0f.astype(jnp.float32)
                wx = xs - x0f.astype(jnp.float32)
                y0 = jnp.clip(y0f, 0, hm1i)
                x0 = jnp.clip(x0f, 0, hm1i)
                y1i = jnp.clip(y0f + 1, 0, hm1i)
                x1i = jnp.clip(x0f + 1, 0, hm1i)
                rbase = base + y0 * wdim
                rbase1 = base + y1i * wdim
                corners = (rbase + x0, rbase + x1i, rbase1 + x0, rbase1 + x1i)
                for k in range(4):
                    idxv[pl.ds(PIXPAD * k + 16 * c, 16)] = corners[k]
                wyv = wy * valid
                w11 = wyv * wx
                w10 = wyv - w11
                wxv = wx * valid
                w01 = wxv - w11
                w00 = valid - wyv - w01
                wbuf[pl.ds(64 * c, 16)] = w00
                wbuf[pl.ds(64 * c + 16, 16)] = w01
                wbuf[pl.ds(64 * c + 32, 16)] = w10
                wbuf[pl.ds(64 * c + 48, 16)] = w11

        def fire_gathers(b):
            idxv, gvb, gsem, _ = bufs[b]
            for h in range(2):
                pltpu.async_copy(
                    table_h.at[idxv.at[pl.ds(2 * PIXPAD * h, 2 * PIXPAD)]],
                    gvb.at[pl.ds(2 * PIXPAD * h, 2 * PIXPAD)], gsem)

        def wait_gathers(b):
            idxv, gvb, gsem, _ = bufs[b]
            for h in range(2):
                pltpu.make_async_copy(
                    table_h.at[idxv.at[pl.ds(2 * PIXPAD * h, 2 * PIXPAD)]],
                    gvb.at[pl.ds(2 * PIXPAD * h, 2 * PIXPAD)], gsem).wait()

        def blend(b):
            gvb = bufs[b][1]
            wbuf = bufs[b][3]
            for c in range(4):
                w00v = wbuf[pl.ds(64 * c, 16)]
                w01v = wbuf[pl.ds(64 * c + 16, 16)]
                w10v = wbuf[pl.ds(64 * c + 32, 16)]
                w11v = wbuf[pl.ds(64 * c + 48, 16)]
                for l in range(16):
                    p = 16 * c + l
                    if p >= NPIX:
                        break
                    w00 = w00v[l]
                    w01 = w01v[l]
                    w10 = w10v[l]
                    w11 = w11v[l]

                    @plsc.parallel_loop(0, CH, 16, unroll=2)
                    def _chunk(s0, p=p, w00=w00, w01=w01, w10=w10, w11=w11):
                        s = pl.ds(s0, 16)
                        ov[p, s] = (
                            gvb[p, s] * w00
                            + gvb[PIXPAD + p, s] * w01
                            + gvb[2 * PIXPAD + p, s] * w10
                            + gvb[3 * PIXPAD + p, s] * w11)

        def fire_out(i):
            pltpu.async_copy(ov, out_h.at[base_box + i], osem)

        def wait_out(i):
            pltpu.make_async_copy(ov, out_h.at[base_box + i], osem).wait()

        prep_box(0, 0)
        fire_gathers(0)

        def pair_body(j, carry):
            i0 = 2 * j
            i1 = i0 + 1
            prep_box(i1, 1)
            fire_gathers(1)
            wait_gathers(0)

            @pl.when(j >= 1)
            def _():
                wait_out(i0 - 1)
            blend(0)
            fire_out(i0)

            @pl.when(jnp.logical_and((i0 + 2) & (CLSHALF - 1) == 0,
                                     i0 + 2 < nb))
            def _():
                pltpu.sync_copy(
                    cls_h.at[pl.ds((base_box + i0 + 2) * CLSPAD,
                                   CLSHALF * CLSPAD)], clsv)

            @pl.when(i0 + 2 < nb)
            def _():
                prep_box(i0 + 2, 0)
                fire_gathers(0)
            wait_gathers(1)
            wait_out(i0)
            blend(1)
            fire_out(i1)
            return carry

        lax.fori_loop(0, nb // 2, pair_body, 0)
        wait_out(nb - 1)

    return sc_kernel(table, boxesf, clsf)


def kernel(P3, P4, P5, P6, boxes, nms_classification):
    table = jnp.concatenate(
        [P3.reshape(4096, CH), P4.reshape(1024, CH),
         P5.reshape(256, CH), P6.reshape(64, CH)], axis=0)
    boxesf = jnp.pad(boxes.reshape(NBOX, 4),
                     ((0, NBOXPAD - NBOX), (0, BOXREC - 4))).reshape(
                         NBOXPAD * BOXREC)
    cls = nms_classification.reshape(NBOX, CLS)
    clsf = jnp.pad(cls, ((0, NBOXPAD - NBOX), (0, CLSPAD - CLS)),
                   constant_values=-jnp.inf).reshape(NBOXPAD * CLSPAD)
    out = _roialign_call(table, boxesf, clsf)
    return out.reshape(1, NBOX, POOL, POOL, CH)


# speedup vs baseline: 1.0950x; 1.0432x over previous
import functools
import jax
import jax.numpy as jnp
from jax import lax
from jax.experimental import pallas as pl
from jax.experimental.pallas import tpu as pltpu
from jax.experimental.pallas import tpu_sc as plsc

POOL = 7
NPIX = POOL * POOL
PIXPAD = 56
CH = 256
OUTROW = NPIX * CH
NBOX = 1000
NBOXPAD = 1024
BOX_PER_W = 32
BOXREC = 8
CLS = 81
CLSPAD = 96
CLSHALF = 4
IDXBUF = 4 * PIXPAD + 8
A0 = (224.0 / 512.0) ** 2 / 2.0
A1 = (224.0 / 512.0) ** 2 * 2.0
A2 = (224.0 / 512.0) ** 2 * 8.0
B3, B4, B5, B6 = 0, 4096, 5120, 5376


def _roialign_call(table, boxesf, clsf):
    mesh = plsc.VectorSubcoreMesh(core_axis_name="c", subcore_axis_name="s")

    @functools.partial(
        pl.kernel,
        out_type=jax.ShapeDtypeStruct((NBOX, NPIX, CH), jnp.float32),
        mesh=mesh,
        scratch_types=[
            pltpu.VMEM((BOX_PER_W * BOXREC,), jnp.float32),
            pltpu.VMEM((CLSHALF * CLSPAD,), jnp.float32),
            pltpu.VMEM((IDXBUF,), jnp.int32),
            pltpu.VMEM((IDXBUF,), jnp.int32),
            pltpu.VMEM((NPIX, CH), jnp.float32),
            pltpu.VMEM((NPIX, CH), jnp.float32),
            pltpu.VMEM((NPIX, CH), jnp.float32),
            pltpu.VMEM((NPIX, CH), jnp.float32),
            pltpu.VMEM((NPIX, CH), jnp.float32),
            pltpu.VMEM((NPIX, CH), jnp.float32),
            pltpu.VMEM((NPIX, CH), jnp.float32),
            pltpu.VMEM((NPIX, CH), jnp.float32),
            pltpu.VMEM((NPIX, CH), jnp.float32),
            pltpu.VMEM((256,), jnp.float32),
            pltpu.VMEM((256,), jnp.float32),
            pltpu.SemaphoreType.DMA,
            pltpu.SemaphoreType.DMA,
            pltpu.SemaphoreType.DMA,
        ],
    )
    def sc_kernel(table_h, boxes_h, cls_h, out_h,
                  boxv, clsv, idx0, idx1,
                  g00, g01, g02, g03, g10, g11, g12, g13,
                  ov, wb0, wb1, gsem0, gsem1, osem):
        bufs = ((idx0, (g00, g01, g02, g03), gsem0, wb0),
                (idx1, (g10, g11, g12, g13), gsem1, wb1))
        wid = lax.axis_index("s") * 2 + lax.axis_index("c")
        base_box = wid * BOX_PER_W
        nb = jnp.minimum(BOX_PER_W, NBOX - base_box)

        pltpu.sync_copy(
            boxes_h.at[pl.ds(base_box * BOXREC, BOX_PER_W * BOXREC)], boxv)
        pltpu.sync_copy(
            cls_h.at[pl.ds(base_box * CLSPAD, CLSHALF * CLSPAD)], clsv)

        lanes = lax.iota(jnp.int32, 16)
        lane0 = lanes == 0
        neg_inf = jnp.full((16,), -jnp.inf, jnp.float32)
        perms = [lanes ^ k for k in (1, 2, 4, 8)]

        def prep_box(i, b):
            idxv = bufs[b][0]
            wbuf = bufs[b][3]
            bv = boxv[pl.ds(i * BOXREC, 16)]
            y1 = bv[0]
            x1 = bv[1]
            y2 = bv[2]
            x2 = bv[3]
            dy = y2 - y1
            dx = x2 - x1
            area = dy * dx
            l3 = area < A0
            l4 = area < A1
            l5 = area < A2
            hm1f = jnp.where(l3, 63.0, jnp.where(l4, 31.0,
                             jnp.where(l5, 15.0, 7.0))).astype(jnp.float32)
            hm1i = jnp.where(l3, 63, jnp.where(l4, 31, jnp.where(l5, 15, 7)))
            wdim = jnp.where(l3, 64, jnp.where(l4, 32, jnp.where(l5, 16, 8)))
            base = jnp.where(l3, B3, jnp.where(l4, B4, jnp.where(l5, B5, B6)))

            cb = (i & (CLSHALF - 1)) * CLSPAD
            c0 = clsv[pl.ds(cb, 16)]
            m = jnp.where(lane0, neg_inf, c0)
            for cchunk in range(1, 6):
                m = jnp.maximum(m, clsv[pl.ds(cb + 16 * cchunk, 16)])
            for perm in perms:
                m = jnp.maximum(m, jnp.take(m, perm))
            valid = (m[0] > c0[0]).astype(jnp.float32)

            wvecs = []
            for c in (3, 0, 1, 2):
                p = lanes + 16 * c
                pi = (p * 9363) >> 16
                pj = p - pi * POOL
                ly = pi.astype(jnp.float32) * jnp.float32(1.0 / 6.0)
                lx = pj.astype(jnp.float32) * jnp.float32(1.0 / 6.0)
                ys = (y1 + dy * ly) * hm1f
                xs = (x1 + dx * lx) * hm1f
                y0f = ys.astype(jnp.int32)
                x0f = xs.astype(jnp.int32)
                wy = ys - y0f.astype(jnp.float32)
                wx = xs - x0f.astype(jnp.float32)
                y0 = jnp.clip(y0f, 0, hm1i)
                x0 = jnp.clip(x0f, 0, hm1i)
                y1i = jnp.clip(y0f + 1, 0, hm1i)
                x1i = jnp.clip(x0f + 1, 0, hm1i)
                rbase = base + y0 * wdim
                rbase1 = base + y1i * wdim
                corners = (rbase + x0, rbase + x1i, rbase1 + x0, rbase1 + x1i)
                for k in range(4):
                    idxv[pl.ds(PIXPAD * k + 16 * c, 16)] = corners[k]
                wyv = wy * valid
                w11 = wyv * wx
                w10 = wyv - w11
                wxv = wx * valid
                w01 = wxv - w11
                w00 = valid - wyv - w01
                wbuf[pl.ds(64 * c, 16)] = w00
                wbuf[pl.ds(64 * c + 16, 16)] = w01
                wbuf[pl.ds(64 * c + 32, 16)] = w10
                wbuf[pl.ds(64 * c + 48, 16)] = w11

        def fire_gathers(b):
            idxv, gvb, gsem, _ = bufs[b]
            for k in range(4):
                pltpu.async_copy(
                    table_h.at[idxv.at[pl.ds(PIXPAD * k, NPIX)]],
                    gvb[k], gsem)

        def wait_gathers(b):
            idxv, gvb, gsem, _ = bufs[b]
            for k in range(4):
                pltpu.make_async_copy(
                    table_h.at[idxv.at[pl.ds(PIXPAD * k, NPIX)]],
                    gvb[k], gsem).wait()

        def blend(b):
            gvb = bufs[b][1]
            wbuf = bufs[b][3]
            for c in range(4):
                w00v = wbuf[pl.ds(64 * c, 16)]
                w01v = wbuf[pl.ds(64 * c + 16, 16)]
                w10v = wbuf[pl.ds(64 * c + 32, 16)]
                w11v = wbuf[pl.ds(64 * c + 48, 16)]
                for l in range(16):
                    p = 16 * c + l
                    if p >= NPIX:
                        break
                    w00 = w00v[l]
                    w01 = w01v[l]
                    w10 = w10v[l]
                    w11 = w11v[l]

                    @plsc.parallel_loop(0, CH, 16, unroll=2)
                    def _chunk(s0, p=p, w00=w00, w01=w01, w10=w10, w11=w11):
                        s = pl.ds(s0, 16)
                        ov[p, s] = (
                            gvb[0][p, s] * w00 + gvb[1][p, s] * w01
                            + gvb[2][p, s] * w10 + gvb[3][p, s] * w11)

        def fire_out(i):
            pltpu.async_copy(ov, out_h.at[base_box + i], osem)

        def wait_out(i):
            pltpu.make_async_copy(ov, out_h.at[base_box + i], osem).wait()

        prep_box(0, 0)
        fire_gathers(0)

        def pair_body(j, carry):
            i0 = 2 * j
            i1 = i0 + 1
            prep_box(i1, 1)
            fire_gathers(1)
            wait_gathers(0)

            @pl.when(j >= 1)
            def _():
                wait_out(i0 - 1)
            blend(0)
            fire_out(i0)

            @pl.when(jnp.logical_and((i0 + 2) & (CLSHALF - 1) == 0,
                                     i0 + 2 < nb))
            def _():
                pltpu.sync_copy(
                    cls_h.at[pl.ds((base_box + i0 + 2) * CLSPAD,
                                   CLSHALF * CLSPAD)], clsv)

            @pl.when(i0 + 2 < nb)
            def _():
                prep_box(i0 + 2, 0)
                fire_gathers(0)
            wait_gathers(1)
            wait_out(i0)
            blend(1)
            fire_out(i1)
            return carry

        lax.fori_loop(0, nb // 2, pair_body, 0)
        wait_out(nb - 1)

    return sc_kernel(table, boxesf, clsf)


def kernel(P3, P4, P5, P6, boxes, nms_classification):
    table = jnp.concatenate(
        [P3.reshape(4096, CH), P4.reshape(1024, CH),
         P5.reshape(256, CH), P6.reshape(64, CH)], axis=0)
    boxesf = jnp.pad(boxes.reshape(NBOX, 4),
                     ((0, NBOXPAD - NBOX), (0, BOXREC - 4))).reshape(
                         NBOXPAD * BOXREC)
    cls = nms_classification.reshape(NBOX, CLS)
    clsf = jnp.pad(cls, ((0, NBOXPAD - NBOX), (0, CLSPAD - CLS)),
                   constant_values=-jnp.inf).reshape(NBOXPAD * CLSPAD)
    out = _roialign_call(table, boxesf, clsf)
    return out.reshape(1, NBOX, POOL, POOL, CH)
